# baseline (speedup 1.0000x reference)
import functools

import jax
import jax.numpy as jnp
from jax import lax
from jax.experimental import pallas as pl
from jax.experimental.pallas import tpu as pltpu
from jax.experimental.pallas import tpu_sc as plsc

_MAX_LENGTH = 2048
_SEQ = 2048
_NROWS = 2 * _MAX_LENGTH - 1
_D = 32
_NC = 2
_NS = 16
_NW = _NC * _NS
_ROWS_PER_W = _SEQ // _NW
_DEPTH = 4


def _sc_expand(revT8, bt):
    mesh = plsc.VectorSubcoreMesh(core_axis_name="c", subcore_axis_name="s")

    @functools.partial(
        pl.kernel,
        mesh=mesh,
        out_type=[
            jax.ShapeDtypeStruct((_SEQ, 4, 16, 8, 128), jnp.float32),
            jax.ShapeDtypeStruct((_SEQ // 8, 16, 8, 128), jnp.float32),
        ],
        scratch_types=[
            pltpu.VMEM_SHARED((8, _D, 4096), jnp.float32),
            pltpu.VMEM_SHARED((16, 31, 8, 128), jnp.float32),
            pltpu.SemaphoreType.DMA,
            pltpu.SemaphoreType.DMA,
        ],
        compiler_params=pltpu.CompilerParams(use_tc_tiling_on_sc=False),
    )
    def k(revT8_hbm, bt_hbm, out_emb, out_bias, sh_revT8, sh_bt, sem_e, sem_b):
        c = lax.axis_index("c")
        s = lax.axis_index("s")
        wid = s * _NC + c
        base = wid * _ROWS_PER_W

        @pl.when(s == 0)
        def _fill():
            pltpu.sync_copy(revT8_hbm, sh_revT8)
            pltpu.sync_copy(bt_hbm, sh_bt)

        plsc.subcore_barrier()

        def _start_plane(i):
            off = (_SEQ - 1) - i
            p = lax.rem(off, 8)
            m = off - p
            for td in range(4):
                for tj in range(16):
                    src = sh_revT8.at[
                        p, pl.ds(8 * td, 8),
                        pl.ds(pl.multiple_of(m + 128 * tj, 8), 128)]
                    pltpu.make_async_copy(src, out_emb.at[i, td, tj],
                                          sem_e).start()


        def _wait_plane():
            def w(j, _):
                pltpu.make_async_copy(
                    sh_revT8.at[0, pl.ds(0, 8), pl.ds(0, 128)],
                    out_emb.at[0, 0, 0], sem_e).wait()
                return 0
            lax.fori_loop(0, 64, w, 0, unroll=8)

        def _bias_cp(tr):
            a = lax.div(tr, 16)
            b = lax.rem(tr, 16)
            return pltpu.make_async_copy(
                sh_bt.at[b, pl.ds(15 - a, 16)], out_bias.at[tr], sem_b)

        tr_base = wid * (_ROWS_PER_W // 8)

        def body(i, _):
            _start_plane(i)

            @pl.when(lax.rem(i, 8) == 0)
            def _bias_start():
                _bias_cp(tr_base + lax.div(i - base, 8)).start()

            @pl.when(i >= base + 2)
            def _drain_prev():
                _wait_plane()

            return 0

        lax.fori_loop(base, base + _ROWS_PER_W, body, 0)
        _wait_plane()
        _wait_plane()

        def tail(tr, _):
            _bias_cp(tr).wait()
            return 0

        lax.fori_loop(tr_base, tr_base + _ROWS_PER_W // 8, tail, 0)

    return k(revT8, bt)


def kernel(rel_pos_emb, rel_pos_bias, seq_length):
    shift = jnp.asarray(seq_length, jnp.int32) - _SEQ
    emb = jnp.roll(rel_pos_emb, -shift, axis=0)
    bias = jnp.roll(rel_pos_bias, -shift, axis=0)

    revT = jnp.flip(emb, axis=0).T
    revTflat = jnp.pad(revT, ((0, 0), (0, 9))).reshape(-1)
    revT8 = jnp.stack(
        [jnp.concatenate(
            [lax.dynamic_slice_in_dim(revTflat, 4104 * d + p, 4096)
             for d in range(_D)]) for p in range(8)])
    revT8 = revT8.reshape(8, _D, 4096)

    bt = jnp.stack(
        [jnp.stack([lax.dynamic_slice_in_dim(bias, 8 * b + s, 3968)
                    for s in range(8)]) for b in range(16)])
    bt = jnp.flip(bt, axis=2)
    bt = jnp.transpose(bt.reshape(16, 8, 31, 128), (0, 2, 1, 3))

    out5, outb4 = _sc_expand(revT8, bt)
    x = jnp.transpose(out5, (0, 1, 3, 2, 4)).reshape(_SEQ, _D, _SEQ)
    y = jnp.transpose(outb4, (0, 2, 1, 3)).reshape(_SEQ, _SEQ)
    return (jnp.transpose(x, (0, 2, 1)), y)


# speedup vs baseline: 1.1954x; 1.1954x over previous
import functools

import jax
import jax.numpy as jnp
from jax import lax
from jax.experimental import pallas as pl
from jax.experimental.pallas import tpu as pltpu
from jax.experimental.pallas import tpu_sc as plsc

_MAX_LENGTH = 2048
_SEQ = 2048
_NROWS = 2 * _MAX_LENGTH - 1
_D = 32
_NC = 2
_NS = 16
_NW = _NC * _NS
_ROWS_PER_W = _SEQ // _NW
_DEPTH = 4


def _sc_expand(revT8, bt):
    mesh = plsc.VectorSubcoreMesh(core_axis_name="c", subcore_axis_name="s")

    @functools.partial(
        pl.kernel,
        mesh=mesh,
        out_type=[
            jax.ShapeDtypeStruct((_SEQ, 4, 16, 8, 128), jnp.float32),
            jax.ShapeDtypeStruct((_SEQ // 8, 16, 8, 128), jnp.float32),
        ],
        scratch_types=[
            pltpu.VMEM_SHARED((8, _D, 4096), jnp.float32),
            pltpu.VMEM_SHARED((16, 31, 8, 128), jnp.float32),
            pltpu.SemaphoreType.DMA,
            pltpu.SemaphoreType.DMA,
        ],
        compiler_params=pltpu.CompilerParams(use_tc_tiling_on_sc=False),
    )
    def k(revT8_hbm, bt_hbm, out_emb, out_bias, sh_revT8, sh_bt, sem_e, sem_b):
        c = lax.axis_index("c")
        s = lax.axis_index("s")
        wid = s * _NC + c
        base = wid * _ROWS_PER_W

        @pl.when(s == 0)
        def _fill():
            pltpu.sync_copy(revT8_hbm, sh_revT8)
            pltpu.sync_copy(bt_hbm, sh_bt)

        plsc.subcore_barrier()

        def _start_plane(i):
            off = (_SEQ - 1) - i
            p = lax.rem(off, 8)
            m = off - p
            for td in range(4):
                for tj in range(16):
                    src = sh_revT8.at[
                        p, pl.ds(8 * td, 8),
                        pl.ds(pl.multiple_of(m + 128 * tj, 8), 128)]
                    pltpu.make_async_copy(src, out_emb.at[i, td, tj],
                                          sem_e).start()


        def _wait_plane():
            def w(j, _):
                pltpu.make_async_copy(
                    sh_revT8.at[0, pl.ds(0, 8), pl.ds(0, 128)],
                    out_emb.at[0, 0, 0], sem_e).wait()
                return 0
            lax.fori_loop(0, 64, w, 0, unroll=8)

        def _bias_cp(tr):
            a = lax.div(tr, 16)
            b = lax.rem(tr, 16)
            return pltpu.make_async_copy(
                sh_bt.at[b, pl.ds(15 - a, 16)], out_bias.at[tr], sem_b)

        tr_base = wid * (_ROWS_PER_W // 8)

        def body(i, _):
            _start_plane(i)

            @pl.when(lax.rem(i, 8) == 0)
            def _bias_start():
                _bias_cp(tr_base + lax.div(i - base, 8)).start()

            @pl.when(i >= base + 2)
            def _drain_prev():
                _wait_plane()

            return 0

        lax.fori_loop(base, base + _ROWS_PER_W, body, 0)
        _wait_plane()
        _wait_plane()

        def tail(tr, _):
            _bias_cp(tr).wait()
            return 0

        lax.fori_loop(tr_base, tr_base + _ROWS_PER_W // 8, tail, 0)

    return k(revT8, bt)


def kernel(rel_pos_emb, rel_pos_bias, seq_length):
    shift = jnp.asarray(seq_length, jnp.int32) - _SEQ
    emb = jnp.roll(rel_pos_emb, -shift, axis=0)
    bias = jnp.roll(rel_pos_bias, -shift, axis=0)

    revT = jnp.flip(emb, axis=0).T
    revTpad = jnp.pad(revT, ((0, 0), (0, 9)))
    revT8 = jnp.stack(
        [lax.dynamic_slice_in_dim(revTpad, p, 4096, axis=1) for p in range(8)])

    bt = jnp.stack(
        [jnp.stack([lax.dynamic_slice_in_dim(bias, 8 * b + s, 3968)
                    for s in range(8)]) for b in range(16)])
    bt = jnp.flip(bt, axis=2)
    bt = jnp.transpose(bt.reshape(16, 8, 31, 128), (0, 2, 1, 3))

    out5, outb4 = _sc_expand(revT8, bt)
    x = jnp.transpose(out5, (0, 1, 3, 2, 4)).reshape(_SEQ, _D, _SEQ)
    y = jnp.transpose(outb4, (0, 2, 1, 3)).reshape(_SEQ, _SEQ)
    return (jnp.transpose(x, (0, 2, 1)), y)
